# baseline (speedup 1.0000x reference)
import functools

import jax
import jax.numpy as jnp
from jax import lax
from jax.experimental import pallas as pl
from jax.experimental.pallas import tpu as pltpu
from jax.experimental.pallas import tpu_sc as plsc

B, N, D, NNZ = 3, 10000, 128, 480000
L = 16
NW = 32
EPT = 15008
NNZ_PAD = EPT * NW
CH = 128
NG = CH // L
CHP = 224
NGP = CHP // L
CHH = 1072
ROWS = B * N
NR = 64
RPT = 512
ROWS_PAD = NR * RPT
OFFS_PAD = 80

_mesh = plsc.VectorSubcoreMesh(core_axis_name="c", subcore_axis_name="s")
_params = pltpu.CompilerParams(
    needs_layout_passes=False, disable_bounds_checks=True)


def _hist_kernel():
    @functools.partial(
        pl.kernel,
        mesh=_mesh,
        out_type=jax.ShapeDtypeStruct((NW, NR * L), jnp.int32),
        scratch_types=[
            pltpu.VMEM((NR * L,), jnp.int32),
            pltpu.VMEM((CHH,), jnp.int32),
        ],
        compiler_params=_params,
    )
    def k(dst_hbm, hist_hbm, hist_v, st_v):
        wid = lax.axis_index("s") * 2 + lax.axis_index("c")
        iota = lax.iota(jnp.int32, L)
        zeros = jnp.zeros((L,), jnp.int32)
        ones = jnp.full((L,), 1, jnp.int32)
        for b in range(NR):
            hist_v[pl.ds(b * L, L)] = zeros
        tbase = wid * EPT

        def chunk(c, _):
            pltpu.sync_copy(dst_hbm.at[pl.ds(tbase + c * CHH, CHH)], st_v)
            for v in range(CHH // L):
                dv = st_v[pl.ds(v * L, L)]
                bkt = lax.shift_right_logical(dv, 9)
                plsc.addupdate_scatter(hist_v, [bkt * L + iota], ones)
            return 0
        lax.fori_loop(0, EPT // CHH, chunk, 0)
        pltpu.sync_copy(hist_v, hist_hbm.at[wid])

    return k


def _part_kernel():
    @functools.partial(
        pl.kernel,
        mesh=_mesh,
        out_type=jax.ShapeDtypeStruct((NNZ_PAD,), jnp.int32),
        scratch_types=[
            pltpu.VMEM((NR * L,), jnp.int32),
            pltpu.VMEM((CHP,), jnp.int32),
            pltpu.VMEM((CHP,), jnp.int32),
        ],
        compiler_params=_params,
    )
    def k(dst_hbm, base_hbm, pos_hbm, ctr_v, st_v, pos_v):
        wid = lax.axis_index("s") * 2 + lax.axis_index("c")
        iota = lax.iota(jnp.int32, L)
        pltpu.sync_copy(base_hbm.at[wid], ctr_v)
        tbase = wid * EPT

        def chunk(c, _):
            pltpu.sync_copy(dst_hbm.at[pl.ds(tbase + c * CHP, CHP)], st_v)
            for g in range(NGP):
                dv = st_v[pl.ds(g * L, L)]
                adr = lax.shift_right_logical(dv, 9) * L + iota
                pos = plsc.load_gather(ctr_v, [adr])
                plsc.store_scatter(ctr_v, [adr], pos + 1)
                pos_v[pl.ds(g * L, L)] = pos
            pltpu.sync_copy(pos_v, pos_hbm.at[pl.ds(tbase + c * CHP, CHP)])
            return 0
        lax.fori_loop(0, EPT // CHP, chunk, 0)

    return k


def _agg_kernel():
    @functools.partial(
        pl.kernel,
        mesh=_mesh,
        out_type=jax.ShapeDtypeStruct((ROWS_PAD * D,), jnp.float32),
        scratch_types=[
            pltpu.VMEM((OFFS_PAD,), jnp.int32),
            pltpu.VMEM((3, CH), jnp.int32),
            pltpu.VMEM((3, CH), jnp.int32),
            pltpu.VMEM((CH, D), jnp.float32),
            pltpu.VMEM((CH, D), jnp.float32),
            pltpu.VMEM((RPT * D,), jnp.float32),
            pltpu.SemaphoreType.DMA,
            pltpu.SemaphoreType.DMA,
        ],
        compiler_params=_params,
    )
    def k(x_hbm, ed_hbm, offs_hbm, out_hbm,
          offs_v, ed0_v, ed1_v, buf0_v, buf1_v, acc_v,
          sem0, sem1):
        sems = (sem0, sem1)
        eds = (ed0_v, ed1_v)
        bufs = (buf0_v, buf1_v)
        wid = lax.axis_index("s") * 2 + lax.axis_index("c")
        pltpu.sync_copy(offs_hbm, offs_v)
        iota = lax.iota(jnp.int32, L)
        zeros16 = jnp.zeros((L,), jnp.float32)
        zero = jnp.zeros((L,), jnp.int32)
        erow = [jnp.full((L,), g * L, jnp.int32) + iota for g in range(NG)]

        def issue(c, s):
            pltpu.sync_copy(ed_hbm.at[:, pl.ds(c * CH, CH)], eds[s])
            pltpu.async_copy(x_hbm.at[eds[s].at[0]], bufs[s], sems[s])

        def wait(s):
            pltpu.make_async_copy(x_hbm.at[eds[s].at[0]], bufs[s],
                                  sems[s]).wait()

        for p in range(2):
            r = p * 32 + wid
            dstbase = r * RPT

            def zbody(i, _):
                for j in range(8):
                    acc_v[pl.ds(i * D + j * L, L)] = zeros16
                return 0
            lax.fori_loop(0, RPT, zbody, 0)

            sel = jnp.full((L,), r, jnp.int32) + jnp.where(iota == 1, 1, 0)
            ov = plsc.load_gather(offs_v, [sel])
            e0 = jnp.sum(jnp.where(iota == 0, ov, 0))
            e1 = jnp.sum(jnp.where(iota == 1, ov, 0))
            c0 = e0 // CH
            c1 = (e1 + CH - 1) // CH

            one = jnp.full((L,), 1, jnp.int32)
            two = jnp.full((L,), 2, jnp.int32)

            def compute(c, s):
                base = c * CH
                vgs = []
                rbs = []
                for g in range(NG):
                    gi = jnp.full((L,), g * L, jnp.int32) + iota + base
                    m = (gi >= e0) & (gi < e1)
                    vg = plsc.bitcast(
                        plsc.load_gather(eds[s], [two, erow[g]]),
                        jnp.float32)
                    vgs.append(jnp.where(m, vg, 0.0))
                    dg = plsc.load_gather(eds[s], [one, erow[g]]) - dstbase
                    rbs.append(jnp.where(m, dg, 0) * D)

                def dbody(d, w):
                    xs = [plsc.load_gather(bufs[s], [erow[g], w]) * vgs[g]
                          for g in range(NG)]
                    for g in range(NG):
                        plsc.addupdate_scatter(acc_v, [rbs[g] + w], xs[g])
                    w = w + 1
                    return jnp.where(w == D, 0, w)
                lax.fori_loop(0, D, dbody, iota)

            @pl.when(c1 > c0)
            def _():
                issue(c0, 0)

            def pair(i, _):
                cA = c0 + 2 * i
                cB = cA + 1

                @pl.when(cB < c1)
                def _():
                    issue(cB, 1)
                wait(0)
                compute(cA, 0)

                @pl.when(cB + 1 < c1)
                def _():
                    issue(cB + 1, 0)

                @pl.when(cB < c1)
                def _():
                    wait(1)
                    compute(cB, 1)
                return 0
            lax.fori_loop(0, (c1 - c0 + 1) // 2, pair, 0)

            pltpu.sync_copy(acc_v, out_hbm.at[pl.ds(r * (RPT * D), RPT * D)])

    return k


_hist = _hist_kernel()
_part = _part_kernel()
_agg = _agg_kernel()


@jax.jit
def kernel(A_batch, A_row, A_col, A_val, X, X_mask, tar_mask):
    n = jnp.int32(N)
    dst = A_batch * n + A_row
    src = A_batch * n + A_col
    pad = NNZ_PAD - NNZ
    dst_p = jnp.concatenate(
        [dst, jnp.full((pad,), ROWS_PAD - 1, jnp.int32)])
    src_p = jnp.concatenate([src, jnp.zeros((pad,), jnp.int32)])
    val_p = jnp.concatenate(
        [lax.bitcast_convert_type(A_val, jnp.int32),
         jnp.zeros((pad,), jnp.int32)])
    hist = _hist(dst_p).reshape(NW, NR, L)
    flat = hist.transpose(1, 0, 2).reshape(-1)
    starts = jnp.concatenate(
        [jnp.zeros((1,), jnp.int32),
         jnp.cumsum(flat, dtype=jnp.int32)[:-1]])
    base = starts.reshape(NR, NW, L).transpose(1, 0, 2).reshape(NW, NR * L)
    bucket_counts = flat.reshape(NR, NW * L).sum(axis=1, dtype=jnp.int32)
    offs = jnp.concatenate(
        [jnp.zeros((1,), jnp.int32),
         jnp.cumsum(bucket_counts, dtype=jnp.int32)])
    offs = jnp.concatenate(
        [offs, jnp.full((OFFS_PAD - NR - 1,), NNZ_PAD, jnp.int32)])

    pos = _part(dst_p, base)
    ed3 = jnp.stack(
        [jnp.zeros((NNZ_PAD,), jnp.int32).at[pos].set(
            a, unique_indices=True)
         for a in (src_p, dst_p, val_p)], axis=0)
    xf = X.reshape(ROWS, D)
    out = _agg(xf, ed3, offs)
    return out[: ROWS * D].reshape(B, N, D)


# speedup vs baseline: 8.4384x; 8.4384x over previous
import functools

import jax
import jax.numpy as jnp
from jax import lax
from jax.experimental import pallas as pl
from jax.experimental.pallas import tpu as pltpu
from jax.experimental.pallas import tpu_sc as plsc

B, N, D, NNZ = 3, 10000, 128, 480000
L = 16
NW = 32
EPT = 15008
NNZ_PAD = EPT * NW
CH = 128
NG = CH // L
CHP = 112
HALF = 240128
NGP = CHP // L
CHH = 1072
ROWS = B * N
NR = 64
RPT = 512
ROWS_PAD = NR * RPT
OFFS_PAD = 80

_mesh = plsc.VectorSubcoreMesh(core_axis_name="c", subcore_axis_name="s")
_params = pltpu.CompilerParams(
    needs_layout_passes=False, disable_bounds_checks=True)


def _hist_kernel():
    @functools.partial(
        pl.kernel,
        mesh=_mesh,
        out_type=jax.ShapeDtypeStruct((NW, NR * L), jnp.int32),
        scratch_types=[
            pltpu.VMEM((NR * L,), jnp.int32),
            pltpu.VMEM((CHH,), jnp.int32),
        ],
        compiler_params=_params,
    )
    def k(dst_hbm, hist_hbm, hist_v, st_v):
        wid = lax.axis_index("c") * 16 + lax.axis_index("s")
        iota = lax.iota(jnp.int32, L)
        zeros = jnp.zeros((L,), jnp.int32)
        ones = jnp.full((L,), 1, jnp.int32)
        for b in range(NR):
            hist_v[pl.ds(b * L, L)] = zeros
        tbase = wid * EPT

        def chunk(c, _):
            pltpu.sync_copy(dst_hbm.at[pl.ds(tbase + c * CHH, CHH)], st_v)
            for v in range(CHH // L):
                dv = st_v[pl.ds(v * L, L)]
                bkt = lax.shift_right_logical(dv, 9)
                plsc.addupdate_scatter(hist_v, [bkt * L + iota], ones)
            return 0
        lax.fori_loop(0, EPT // CHH, chunk, 0)
        pltpu.sync_copy(hist_v, hist_hbm.at[wid])

    return k


def _part_kernel():
    @functools.partial(
        pl.kernel,
        mesh=_mesh,
        out_type=(jax.ShapeDtypeStruct((NNZ_PAD,), jnp.int32),
                  jax.ShapeDtypeStruct((NNZ_PAD,), jnp.int32),
                  jax.ShapeDtypeStruct((NNZ_PAD,), jnp.int32)),
        scratch_types=[
            pltpu.VMEM((NR * L,), jnp.int32),
            pltpu.VMEM((CHP,), jnp.int32),
            pltpu.VMEM((CHP,), jnp.int32),
            pltpu.VMEM((CHP,), jnp.int32),
            pltpu.VMEM((CHP,), jnp.int32),
            pltpu.VMEM_SHARED((HALF,), jnp.int32),
            pltpu.VMEM_SHARED((HALF,), jnp.int32),
            pltpu.VMEM_SHARED((HALF,), jnp.int32),
        ],
        compiler_params=_params,
    )
    def k(src_hbm, dst_hbm, val_hbm, base_hbm, os_hbm, od_hbm, ov_hbm,
          ctr_v, sts_v, std_v, stv_v, pos_v, sh_s, sh_d, sh_v):
        sc = lax.axis_index("c")
        ts = lax.axis_index("s")
        wid = sc * 16 + ts
        iota = lax.iota(jnp.int32, L)
        pltpu.sync_copy(base_hbm.at[wid], ctr_v)
        tbase = wid * EPT

        def chunk(c, _):
            cb = tbase + c * CHP
            pltpu.sync_copy(src_hbm.at[pl.ds(cb, CHP)], sts_v)
            pltpu.sync_copy(dst_hbm.at[pl.ds(cb, CHP)], std_v)
            pltpu.sync_copy(val_hbm.at[pl.ds(cb, CHP)], stv_v)
            for g in range(NGP):
                dv = std_v[pl.ds(g * L, L)]
                adr = lax.shift_right_logical(dv, 9) * L + iota
                pos = plsc.load_gather(ctr_v, [adr])
                plsc.store_scatter(ctr_v, [adr], pos + 1)
                pos_v[pl.ds(g * L, L)] = pos
            pltpu.sync_copy(sts_v, sh_s.at[pos_v])
            pltpu.sync_copy(std_v, sh_d.at[pos_v])
            pltpu.sync_copy(stv_v, sh_v.at[pos_v])
            return 0
        lax.fori_loop(0, EPT // CHP, chunk, 0)

        plsc.subcore_barrier()
        full = 117 * 128
        outs = (os_hbm, od_hbm, ov_hbm)
        for f, sh in enumerate((sh_s, sh_d, sh_v)):
            pltpu.sync_copy(
                sh.at[pl.ds(ts * full, full)],
                outs[f].at[pl.ds(sc * HALF + ts * full, full)])

        @pl.when(ts < 4)
        def _():
            rem = 16 * full
            for f, sh in enumerate((sh_s, sh_d, sh_v)):
                pltpu.sync_copy(
                    sh.at[pl.ds(rem + ts * 128, 128)],
                    outs[f].at[pl.ds(sc * HALF + rem + ts * 128, 128)])

    return k


def _agg_kernel():
    @functools.partial(
        pl.kernel,
        mesh=_mesh,
        out_type=jax.ShapeDtypeStruct((ROWS_PAD * D,), jnp.float32),
        scratch_types=[
            pltpu.VMEM((2 * OFFS_PAD,), jnp.int32),
            pltpu.VMEM((CH,), jnp.int32),
            pltpu.VMEM((CH,), jnp.int32),
            pltpu.VMEM((CH,), jnp.int32),
            pltpu.VMEM((CH,), jnp.int32),
            pltpu.VMEM((CH,), jnp.int32),
            pltpu.VMEM((CH,), jnp.int32),
            pltpu.VMEM((CH, D), jnp.float32),
            pltpu.VMEM((CH, D), jnp.float32),
            pltpu.VMEM((RPT * D,), jnp.float32),
            pltpu.SemaphoreType.DMA,
            pltpu.SemaphoreType.DMA,
        ],
        compiler_params=_params,
    )
    def k(x_hbm, s_hbm, d_hbm, v_hbm, offs_hbm, out_hbm,
          offs_v, es0_v, es1_v, ed0_v, ed1_v, ev0_v, ev1_v,
          buf0_v, buf1_v, acc_v, sem0, sem1):
        sems = (sem0, sem1)
        ess = (es0_v, es1_v)
        eds = (ed0_v, ed1_v)
        evs = (ev0_v, ev1_v)
        bufs = (buf0_v, buf1_v)
        wid = lax.axis_index("s") * 2 + lax.axis_index("c")
        pltpu.sync_copy(offs_hbm, offs_v)
        iota = lax.iota(jnp.int32, L)
        zeros16 = jnp.zeros((L,), jnp.float32)
        zero = jnp.zeros((L,), jnp.int32)
        erow = [jnp.full((L,), g * L, jnp.int32) + iota for g in range(NG)]

        def issue(c, s):
            pltpu.sync_copy(s_hbm.at[pl.ds(c * CH, CH)], ess[s])
            pltpu.sync_copy(d_hbm.at[pl.ds(c * CH, CH)], eds[s])
            pltpu.sync_copy(v_hbm.at[pl.ds(c * CH, CH)], evs[s])
            pltpu.async_copy(x_hbm.at[ess[s]], bufs[s], sems[s])

        def wait(s):
            pltpu.make_async_copy(x_hbm.at[ess[s]], bufs[s],
                                  sems[s]).wait()

        for p in range(2):
            r = p * 32 + wid
            dstbase = r * RPT

            def zbody(i, _):
                for j in range(8):
                    acc_v[pl.ds(i * D + j * L, L)] = zeros16
                return 0
            lax.fori_loop(0, RPT, zbody, 0)

            one = jnp.full((L,), 1, jnp.int32)
            two = jnp.full((L,), 2, jnp.int32)

            for h in range(2):
                sel = (jnp.full((L,), h * OFFS_PAD + r, jnp.int32)
                       + jnp.where(iota == 1, 1, 0))
                ov = plsc.load_gather(offs_v, [sel])
                e0 = jnp.sum(jnp.where(iota == 0, ov, 0)) + h * HALF
                e1 = jnp.sum(jnp.where(iota == 1, ov, 0)) + h * HALF
                c0 = e0 // CH
                c1 = (e1 + CH - 1) // CH

                def compute(c, s):
                    base = c * CH
                    vgs = []
                    rbs = []
                    for g in range(NG):
                        gi = jnp.full((L,), g * L, jnp.int32) + iota + base
                        m = (gi >= e0) & (gi < e1)
                        vg = plsc.bitcast(
                            evs[s][pl.ds(g * L, L)], jnp.float32)
                        vgs.append(jnp.where(m, vg, 0.0))
                        dg = eds[s][pl.ds(g * L, L)] - dstbase
                        rbs.append(jnp.where(m, dg, 0) * D)

                    def dbody(d, w):
                        xs = [plsc.load_gather(bufs[s], [erow[g], w])
                              * vgs[g] for g in range(NG)]
                        for g in range(NG):
                            plsc.addupdate_scatter(
                                acc_v, [rbs[g] + w], xs[g])
                        w = w + 1
                        return jnp.where(w == D, 0, w)
                    lax.fori_loop(0, D, dbody, iota)

                @pl.when(c1 > c0)
                def _():
                    issue(c0, 0)

                def pair(i, _):
                    cA = c0 + 2 * i
                    cB = cA + 1

                    @pl.when(cB < c1)
                    def _():
                        issue(cB, 1)
                    wait(0)
                    compute(cA, 0)

                    @pl.when(cB + 1 < c1)
                    def _():
                        issue(cB + 1, 0)

                    @pl.when(cB < c1)
                    def _():
                        wait(1)
                        compute(cB, 1)
                    return 0
                lax.fori_loop(0, (c1 - c0 + 1) // 2, pair, 0)

            pltpu.sync_copy(acc_v, out_hbm.at[pl.ds(r * (RPT * D), RPT * D)])

    return k


_hist = _hist_kernel()
_part = _part_kernel()
_agg = _agg_kernel()


@jax.jit
def kernel(A_batch, A_row, A_col, A_val, X, X_mask, tar_mask):
    n = jnp.int32(N)
    dst = A_batch * n + A_row
    src = A_batch * n + A_col
    pad = NNZ_PAD - NNZ
    dst_p = jnp.concatenate(
        [dst, jnp.full((pad,), ROWS_PAD - 1, jnp.int32)])
    src_p = jnp.concatenate([src, jnp.zeros((pad,), jnp.int32)])
    val_p = jnp.concatenate(
        [lax.bitcast_convert_type(A_val, jnp.int32),
         jnp.zeros((pad,), jnp.int32)])
    hist = _hist(dst_p).reshape(2, 16, NR, L)
    bases = []
    offs_list = []
    for h in range(2):
        flat = hist[h].transpose(1, 0, 2).reshape(-1)
        starts = jnp.concatenate(
            [jnp.zeros((1,), jnp.int32),
             jnp.cumsum(flat, dtype=jnp.int32)[:-1]])
        bases.append(starts.reshape(NR, 16, L).transpose(1, 0, 2)
                     .reshape(16, NR * L))
        bc = flat.reshape(NR, 16 * L).sum(axis=1, dtype=jnp.int32)
        o = jnp.concatenate(
            [jnp.zeros((1,), jnp.int32),
             jnp.cumsum(bc, dtype=jnp.int32)])
        offs_list.append(jnp.concatenate(
            [o, jnp.full((OFFS_PAD - NR - 1,), HALF, jnp.int32)]))
    base = jnp.concatenate(bases, axis=0)
    offs = jnp.concatenate(offs_list)

    s2, d2, v2 = _part(src_p, dst_p, val_p, base)
    xf = X.reshape(ROWS, D)
    out = _agg(xf, s2, d2, v2, offs)
    return out[: ROWS * D].reshape(B, N, D)
